# baseline (speedup 1.0000x reference)
import jax
import jax.numpy as jnp
from jax import lax
from jax.experimental import pallas as pl
from jax.experimental.pallas import tpu as pltpu
from jax.experimental.pallas import tpu_sc as plsc

_INDIM = 2048
_OUTDIM = 2048
_FANIN = 32
_NTOK = 2048

_E = _OUTDIM * _FANIN
_NUM_WORKERS = 32
_NHALF = 2
_HALF_ROWS = _OUTDIM // _NHALF
_ROWS_PER_WORKER = _HALF_ROWS // _NUM_WORKERS
_CHUNK_ROWS = 16
_CHUNK_EDGES = _CHUNK_ROWS * _FANIN
_LANES = 16
_NBUF = 2


def _scatter_body(ei_hbm, w_hbm, wout_hbm, wbufs, srcbuf, wvbuf, sems, *,
                  row0):
    wid = lax.axis_index("s") * 2 + lax.axis_index("c")
    iota = lax.iota(jnp.int32, _LANES)
    nchunks = _ROWS_PER_WORKER // _CHUNK_ROWS
    pending = [None] * _NBUF

    zeros16 = jnp.zeros((_LANES,), jnp.float32)
    for buf in range(_NBUF):
        for r in range(_CHUNK_ROWS):
            def _zcol(j, carry, buf=buf, r=r):
                base = j * (_LANES * 8)
                for u in range(8):
                    wbufs[buf, r, pl.ds(base + u * _LANES, _LANES)] = zeros16
                return carry
            lax.fori_loop(0, _INDIM // (_LANES * 8), _zcol, 0)

    for chunk in range(nchunks):
        buf = chunk % _NBUF
        row_local = wid * _ROWS_PER_WORKER + chunk * _CHUNK_ROWS
        edge_base = (row0 + row_local) * _FANIN
        wbuf = wbufs.at[buf]

        ones = iota * 0 + 1
        bufv = iota * 0 + buf

        if pending[buf] is not None:
            pending[buf].wait()
            pending[buf] = None
            for k in range(_FANIN):
                le = iota * _FANIN + k
                src_vec = plsc.load_gather(srcbuf, [bufv, ones, le])
                w_vec = plsc.load_gather(
                    wvbuf, [le + buf * _CHUNK_EDGES])
                plsc.addupdate_scatter(wbuf, [iota, src_vec], -w_vec)

        pltpu.sync_copy(ei_hbm.at[:, pl.ds(edge_base, _CHUNK_EDGES)],
                        srcbuf.at[buf])
        pltpu.sync_copy(w_hbm.at[pl.ds(edge_base, _CHUNK_EDGES)],
                        wvbuf.at[pl.ds(buf * _CHUNK_EDGES, _CHUNK_EDGES)])

        for k in range(_FANIN):
            le = iota * _FANIN + k
            src_vec = plsc.load_gather(srcbuf, [bufv, ones, le])
            w_vec = plsc.load_gather(wvbuf, [le + buf * _CHUNK_EDGES])
            plsc.addupdate_scatter(wbuf, [iota, src_vec], w_vec)

        pending[buf] = pltpu.async_copy(
            wbuf, wout_hbm.at[pl.ds(row_local, _CHUNK_ROWS)], sems.at[buf])

    for p in pending:
        if p is not None:
            p.wait()


def _build_w_half(ei_flat, weight, half):
    mesh = plsc.VectorSubcoreMesh(core_axis_name="c", subcore_axis_name="s")

    def body(ei_hbm, w_hbm, wout_hbm, wbufs, srcbuf, wvbuf, sems):
        _scatter_body(ei_hbm, w_hbm, wout_hbm, wbufs, srcbuf, wvbuf, sems,
                      row0=half * _HALF_ROWS)

    k = pl.kernel(
        body,
        mesh=mesh,
        out_type=jax.ShapeDtypeStruct((_HALF_ROWS, _INDIM), jnp.float32),
        scratch_types=[
            pltpu.VMEM((_NBUF, _CHUNK_ROWS, _INDIM), jnp.float32),
            pltpu.VMEM((_NBUF, 2, _CHUNK_EDGES), jnp.int32),
            pltpu.VMEM((_NBUF * _CHUNK_EDGES,), jnp.float32),
            pltpu.SemaphoreType.DMA((_NBUF,)),
        ],
        compiler_params=pltpu.CompilerParams(needs_layout_passes=False),
    )
    return k(ei_flat, weight)


_BN = 256


def _mm_body(x_ref, w_ref, b_ref, o_ref):
    acc = lax.dot_general(
        x_ref[...], w_ref[...].astype(jnp.bfloat16),
        (((1,), (1,)), ((), ())),
        preferred_element_type=jnp.float32,
    )
    o_ref[...] = acc + b_ref[...]


def _mm_body_acc(prev_ref, x_ref, w_ref, b_ref, o_ref):
    del prev_ref
    _mm_body(x_ref, w_ref, b_ref, o_ref)


def _matmul_half(prev, xb, w_half, bias2d, half):
    off = half * (_HALF_ROWS // _BN)
    grid = (_HALF_ROWS // _BN,)
    common = dict(
        grid=grid,
        out_specs=pl.BlockSpec((_NTOK, _BN), lambda j, off=off: (0, j + off)),
        out_shape=jax.ShapeDtypeStruct((_NTOK, _OUTDIM), jnp.float32),
    )
    in_specs = [
        pl.BlockSpec((_NTOK, _INDIM), lambda j: (0, 0)),
        pl.BlockSpec((_BN, _INDIM), lambda j: (j, 0)),
        pl.BlockSpec((1, _BN), lambda j, off=off: (0, j + off)),
    ]
    if prev is None:
        return pl.pallas_call(
            _mm_body, in_specs=in_specs, **common,
        )(xb, w_half, bias2d)
    return pl.pallas_call(
        _mm_body_acc,
        in_specs=[pl.BlockSpec(memory_space=pl.ANY)] + in_specs,
        input_output_aliases={0: 0},
        **common,
    )(prev, xb, w_half, bias2d)


@jax.jit
def kernel(x, weight, bias, edge_index):
    xb = x.astype(jnp.bfloat16)
    bias2d = bias.reshape(1, _OUTDIM)
    w0 = _build_w_half(edge_index, weight, 0)
    w1 = _build_w_half(edge_index, weight, 1)
    out = _matmul_half(None, xb, w0, bias2d, 0)
    out = _matmul_half(out, xb, w1, bias2d, 1)
    return out
